# baseline (device time: 85651 ns/iter reference)
import jax
import jax.numpy as jnp
from jax import lax
from jax.experimental import pallas as pl
from jax.experimental.pallas import tpu as pltpu

N_CHUNKS = 16


def kernel(A, B):
    m, k = A.shape
    k2, n = B.shape
    assert k == k2
    assert m % N_CHUNKS == 0
    mc = m // N_CHUNKS

    def body(
        a_ref, b_ref, out_ref, b_bf,
        q_send, q_recv, s_send, s_recv,
        q_send_sems, q_recv_sems, s_send_sems, s_recv_sems,
    ):
        my_x = lax.axis_index("x")
        my_y = lax.axis_index("y")
        partner = (1 - my_x, my_y)

        b_bf[...] = b_ref[...].astype(jnp.bfloat16)

        barrier_sem = pltpu.get_barrier_semaphore()
        pl.semaphore_signal(
            barrier_sem, inc=1,
            device_id=(my_x, my_y), device_id_type=pl.DeviceIdType.MESH,
        )
        pl.semaphore_wait(barrier_sem, 1)

        rdmas = []
        for i in range(N_CHUNKS):
            rows = pl.ds(i * mc, mc)
            p = jnp.dot(
                a_ref[rows, :].astype(jnp.bfloat16),
                b_bf[...],
                preferred_element_type=jnp.float32,
            )
            out_ref[rows, :] = p
            amax = jnp.maximum(
                jnp.max(jnp.abs(p), axis=0, keepdims=True), 1e-20
            )
            q = jnp.clip(jnp.round(p * (127.0 / amax)), -127.0, 127.0)
            q_send[rows, :] = q.astype(jnp.int8)
            s_send[i : i + 1, :] = amax * (1.0 / 127.0)

            q_rdma = pltpu.make_async_remote_copy(
                src_ref=q_send.at[rows, :],
                dst_ref=q_recv.at[rows, :],
                send_sem=q_send_sems.at[i],
                recv_sem=q_recv_sems.at[i],
                device_id=(my_x, my_y),
                device_id_type=pl.DeviceIdType.MESH,
            )
            q_rdma.start()
            s_rdma = pltpu.make_async_remote_copy(
                src_ref=s_send.at[i : i + 1, :],
                dst_ref=s_recv.at[i : i + 1, :],
                send_sem=s_send_sems.at[i],
                recv_sem=s_recv_sems.at[i],
                device_id=(my_x, my_y),
                device_id_type=pl.DeviceIdType.MESH,
            )
            s_rdma.start()
            rdmas.append((q_rdma, s_rdma))

        for i in range(N_CHUNKS):
            rows = pl.ds(i * mc, mc)
            q_rdma, s_rdma = rdmas[i]
            s_rdma.wait_recv()
            q_rdma.wait_recv()
            deq = q_recv[rows, :].astype(jnp.float32) * s_recv[i : i + 1, :]
            out_ref[rows, :] = out_ref[rows, :] + deq

        for q_rdma, s_rdma in rdmas:
            q_rdma.wait_send()
            s_rdma.wait_send()

    return pl.pallas_call(
        body,
        out_shape=jax.ShapeDtypeStruct((m, n), jnp.float32),
        in_specs=[
            pl.BlockSpec(memory_space=pltpu.VMEM),
            pl.BlockSpec(memory_space=pltpu.VMEM),
        ],
        out_specs=pl.BlockSpec(memory_space=pltpu.VMEM),
        scratch_shapes=[
            pltpu.VMEM((k, n), jnp.bfloat16),
            pltpu.VMEM((m, n), jnp.int8),
            pltpu.VMEM((m, n), jnp.int8),
            pltpu.VMEM((N_CHUNKS, n), jnp.float32),
            pltpu.VMEM((N_CHUNKS, n), jnp.float32),
            pltpu.SemaphoreType.DMA((N_CHUNKS,)),
            pltpu.SemaphoreType.DMA((N_CHUNKS,)),
            pltpu.SemaphoreType.DMA((N_CHUNKS,)),
            pltpu.SemaphoreType.DMA((N_CHUNKS,)),
        ],
        compiler_params=pltpu.CompilerParams(collective_id=0),
    )(A, B)


# device time: 38476 ns/iter; 2.2261x vs baseline; 2.2261x over previous
import jax
import jax.numpy as jnp
from jax import lax
from jax.experimental import pallas as pl
from jax.experimental.pallas import tpu as pltpu

N_CHUNKS = 8


def kernel(A, B):
    m, k = A.shape
    k2, n = B.shape
    assert k == k2
    assert m % N_CHUNKS == 0
    mc = m // N_CHUNKS

    def body(
        a_hbm, b_hbm, out_ref,
        a_vmem, b_vmem, b_bf,
        q_send, q_recv, s_send, s_recv,
        in_sems, a_sems,
        q_send_sems, q_recv_sems, s_send_sems, s_recv_sems,
    ):
        my_x = lax.axis_index("x")
        my_y = lax.axis_index("y")
        partner = (1 - my_x, my_y)

        b_in = pltpu.make_async_copy(b_hbm, b_vmem, in_sems.at[0])
        b_in.start()
        a_ins = []
        for i in range(N_CHUNKS):
            rows = pl.ds(i * mc, mc)
            a_in = pltpu.make_async_copy(
                a_hbm.at[rows, :], a_vmem.at[rows, :], a_sems.at[i]
            )
            a_in.start()
            a_ins.append(a_in)

        barrier_sem = pltpu.get_barrier_semaphore()
        pl.semaphore_signal(
            barrier_sem, inc=1,
            device_id=partner, device_id_type=pl.DeviceIdType.MESH,
        )
        pl.semaphore_wait(barrier_sem, 1)

        b_in.wait()
        b_bf[...] = b_vmem[...].astype(jnp.bfloat16)

        rdmas = []
        for i in range(N_CHUNKS):
            rows = pl.ds(i * mc, mc)
            a_ins[i].wait()
            p = jnp.dot(
                a_vmem[rows, :].astype(jnp.bfloat16),
                b_bf[...],
                preferred_element_type=jnp.float32,
            )
            out_ref[rows, :] = p
            amax = jnp.maximum(
                jnp.max(jnp.abs(p), axis=0, keepdims=True), 1e-20
            )
            q = jnp.clip(jnp.round(p * (127.0 / amax)), -127.0, 127.0)
            q_send[rows, :] = q.astype(jnp.int8)
            s_send[i : i + 1, :] = amax * (1.0 / 127.0)

            q_rdma = pltpu.make_async_remote_copy(
                src_ref=q_send.at[rows, :],
                dst_ref=q_recv.at[rows, :],
                send_sem=q_send_sems.at[i],
                recv_sem=q_recv_sems.at[i],
                device_id=partner,
                device_id_type=pl.DeviceIdType.MESH,
            )
            q_rdma.start()
            s_rdma = pltpu.make_async_remote_copy(
                src_ref=s_send.at[i : i + 1, :],
                dst_ref=s_recv.at[i : i + 1, :],
                send_sem=s_send_sems.at[i],
                recv_sem=s_recv_sems.at[i],
                device_id=partner,
                device_id_type=pl.DeviceIdType.MESH,
            )
            s_rdma.start()
            rdmas.append((q_rdma, s_rdma))

        for i in range(N_CHUNKS):
            rows = pl.ds(i * mc, mc)
            q_rdma, s_rdma = rdmas[i]
            s_rdma.wait_recv()
            q_rdma.wait_recv()
            deq = q_recv[rows, :].astype(jnp.float32) * s_recv[i : i + 1, :]
            out_ref[rows, :] = out_ref[rows, :] + deq

        for q_rdma, s_rdma in rdmas:
            q_rdma.wait_send()
            s_rdma.wait_send()

    return pl.pallas_call(
        body,
        out_shape=jax.ShapeDtypeStruct((m, n), jnp.float32),
        in_specs=[
            pl.BlockSpec(memory_space=pltpu.MemorySpace.HBM),
            pl.BlockSpec(memory_space=pltpu.MemorySpace.HBM),
        ],
        out_specs=pl.BlockSpec(memory_space=pltpu.MemorySpace.VMEM),
        scratch_shapes=[
            pltpu.VMEM((m, k), jnp.float32),
            pltpu.VMEM((k, n), jnp.float32),
            pltpu.VMEM((k, n), jnp.bfloat16),
            pltpu.VMEM((m, n), jnp.int8),
            pltpu.VMEM((m, n), jnp.int8),
            pltpu.VMEM((N_CHUNKS, n), jnp.float32),
            pltpu.VMEM((N_CHUNKS, n), jnp.float32),
            pltpu.SemaphoreType.DMA((2,)),
            pltpu.SemaphoreType.DMA((N_CHUNKS,)),
            pltpu.SemaphoreType.DMA((N_CHUNKS,)),
            pltpu.SemaphoreType.DMA((N_CHUNKS,)),
            pltpu.SemaphoreType.DMA((N_CHUNKS,)),
            pltpu.SemaphoreType.DMA((N_CHUNKS,)),
        ],
        compiler_params=pltpu.CompilerParams(collective_id=0),
    )(
        pltpu.with_memory_space_constraint(A, pltpu.MemorySpace.HBM),
        pltpu.with_memory_space_constraint(B, pltpu.MemorySpace.HBM),
    )
